# Initial kernel scaffold; baseline (speedup 1.0000x reference)
import jax
import jax.numpy as jnp
from jax.experimental import pallas as pl


def kernel(x, pe):
    raise NotImplementedError("write your pallas kernel here")


# speedup vs baseline: 1.0097x; 1.0097x over previous
import jax
import jax.numpy as jnp
from jax.experimental import pallas as pl


_BLOCK_S = 512


def _bcast_kernel(pe_ref, out_ref):
    blk = pe_ref[...]
    for b in range(out_ref.shape[0]):
        out_ref[b] = blk


def kernel(x, pe):
    batch, seq_len = x.shape
    d_model = pe.shape[1]
    grid = (seq_len // _BLOCK_S,)
    out = pl.pallas_call(
        _bcast_kernel,
        grid=grid,
        in_specs=[pl.BlockSpec((_BLOCK_S, d_model), lambda s: (s, 0))],
        out_specs=pl.BlockSpec((batch, _BLOCK_S, d_model), lambda s: (0, s, 0)),
        out_shape=jax.ShapeDtypeStruct((batch, seq_len, d_model), pe.dtype),
    )(pe[:seq_len])
    return out
